# Initial kernel scaffold; baseline (speedup 1.0000x reference)
import functools

import jax
import jax.numpy as jnp
from jax import lax
from jax.experimental import pallas as pl
from jax.experimental.pallas import tpu as pltpu
from jax.experimental.pallas import tpu_sc as plsc

F32 = jnp.float32
I32 = jnp.int32

N_NODE = 10000
DH = 128
ACC_W = 144
NE = 320000
NC, NS = 2, 16
NW = NC * NS
EPT = NE // NW
CHUNK = 80
NCHUNK = EPT // CHUNK
NGROUP = CHUNK // 16
RPT = N_NODE // NS

L_PAD = 102400
DEC_PT = L_PAD // NW
DEC_NCH = DEC_PT // CHUNK

BR = 1000
GRID = N_NODE // BR

_EPS = 1e-16


def _dot(x, w):
    return jnp.dot(x, w, preferred_element_type=F32)


def _proj6_body(xa, xart, wlp, wrp, wlc, wrc, wlr, wrr,
                olp, orp, olc, orc, olr, orr):
    a = xa[...]
    r = xart[...]
    olp[...] = _dot(a, wlp[...])
    orp[...] = _dot(r, wrp[...])
    olc[...] = _dot(a, wlc[...])
    orc[...] = _dot(a, wrc[...])
    olr[...] = _dot(r, wlr[...])
    orr[...] = _dot(a, wrr[...])


def _proj6(xa, xart, ws):
    row = pl.BlockSpec((BR, DH), lambda i: (i, 0))
    wspec = pl.BlockSpec((DH, DH), lambda i: (0, 0))
    out = jax.ShapeDtypeStruct((N_NODE, DH), F32)
    return pl.pallas_call(
        _proj6_body,
        grid=(GRID,),
        in_specs=[row, row] + [wspec] * 6,
        out_specs=[row] * 6,
        out_shape=[out] * 6,
    )(xa, xart, *ws)


def _gat_out(x, b):
    num = x[0, :, :DH] + x[1, :, :DH]
    den = x[0, :, DH] + x[1, :, DH]
    return num / (den + _EPS)[:, None] + b


def _c1_one_body(pp, b, lw, lb, h_ref, ps, pss):
    g = _gat_out(pp[...], b[...])
    h = g + _dot(g, lw[...]) + lb[...]
    h_ref[...] = h
    ps[...] = jnp.sum(h, axis=0, keepdims=True)
    pss[...] = jnp.sum(h * h, axis=0, keepdims=True)


def _c1_two_body(p1, p2, b1, b2, lw, lb, h_ref, ps, pss):
    g = _gat_out(p1[...], b1[...]) + _gat_out(p2[...], b2[...])
    h = g + _dot(g, lw[...]) + lb[...]
    h_ref[...] = h
    ps[...] = jnp.sum(h, axis=0, keepdims=True)
    pss[...] = jnp.sum(h * h, axis=0, keepdims=True)


def _c1_specs(nparts):
    pspec = pl.BlockSpec((2, BR, ACC_W), lambda i: (0, i, 0))
    vspec = pl.BlockSpec((DH,), lambda i: (0,))
    wspec = pl.BlockSpec((DH, DH), lambda i: (0, 0))
    row = pl.BlockSpec((BR, DH), lambda i: (i, 0))
    st = pl.BlockSpec((1, DH), lambda i: (i, 0))
    in_specs = [pspec] * nparts + [vspec] * nparts + [wspec, vspec]
    out_specs = [row, st, st]
    out_shape = [jax.ShapeDtypeStruct((N_NODE, DH), F32),
                 jax.ShapeDtypeStruct((GRID, DH), F32),
                 jax.ShapeDtypeStruct((GRID, DH), F32)]
    return in_specs, out_specs, out_shape


def _c1_one(part, b, lw, lb):
    i, o, s = _c1_specs(1)
    return pl.pallas_call(_c1_one_body, grid=(GRID,), in_specs=i,
                          out_specs=o, out_shape=s)(part, b, lw, lb)


def _c1_two(part1, part2, b1, b2, lw, lb):
    i, o, s = _c1_specs(2)
    return pl.pallas_call(_c1_two_body, grid=(GRID,), in_specs=i,
                          out_specs=o, out_shape=s)(part1, part2, b1, b2,
                                                    lw, lb)


def _bn_apply(h, ps, pss, g, b):
    mu = jnp.sum(ps[...], axis=0) * (1.0 / N_NODE)
    msq = jnp.sum(pss[...], axis=0) * (1.0 / N_NODE)
    var = msq - mu * mu
    scale = g[...] * lax.rsqrt(var + 1e-5)
    shift = b[...] - mu * scale
    return h[...] * scale + shift


def _c3_body(h, ps, pss, g, b, *rest):
    nw = len(rest) // 2
    y = _bn_apply(h, ps, pss, g, b)
    if nw:
        y = jnp.maximum(y, 0.0)
        for k in range(nw):
            rest[nw + k][...] = _dot(y, rest[k][...])
    else:
        rest[0][...] = y


def _c3(h, ps, pss, g, b, ws):
    nw = len(ws)
    row = pl.BlockSpec((BR, DH), lambda i: (i, 0))
    st = pl.BlockSpec((GRID, DH), lambda i: (0, 0))
    vspec = pl.BlockSpec((DH,), lambda i: (0,))
    wspec = pl.BlockSpec((DH, DH), lambda i: (0, 0))
    nout = max(nw, 1)
    return pl.pallas_call(
        _c3_body,
        grid=(GRID,),
        in_specs=[row, st, st, vspec, vspec] + [wspec] * nw,
        out_specs=[row] * nout,
        out_shape=[jax.ShapeDtypeStruct((N_NODE, DH), F32)] * nout,
    )(h, ps, pss, g, b, *ws)


_MESH = plsc.VectorSubcoreMesh(core_axis_name="c", subcore_axis_name="s")


def _edge_body(xl_hbm, xr_hbm, src_hbm, dst_hbm, att_hbm, out_hbm,
               srcv, dstv, attv, arows, brows, wbuf, zerov, accsh,
               sema, semb):
    cid = lax.axis_index("c")
    sid = lax.axis_index("s")
    wid = cid * NS + sid

    pltpu.sync_copy(src_hbm.at[wid], srcv)
    pltpu.sync_copy(dst_hbm.at[wid], dstv)
    pltpu.sync_copy(att_hbm, attv)

    z16 = jnp.zeros((16,), F32)

    def zb(r, _):
        for c in range(ACC_W // 16):
            zerov[r, pl.ds(c * 16, 16)] = z16
        return 0
    lax.fori_loop(0, NCHUNK, zb, 0)

    def zw(r, _):
        wbuf[r, pl.ds(DH, 16)] = z16
        return 0
    lax.fori_loop(0, CHUNK, zw, 0)

    for k in range(RPT // NCHUNK):
        pltpu.sync_copy(zerov,
                        accsh.at[pl.ds(sid * RPT + k * NCHUNK, NCHUNK)])
    plsc.subcore_barrier()

    rows0 = lax.iota(I32, 16)
    colden = jnp.full((16,), DH, I32)

    def chunk(j, _):
        cpa = pltpu.async_copy(xl_hbm.at[srcv.at[j]], arows, sema)
        cpb = pltpu.async_copy(xr_hbm.at[dstv.at[j]], brows, semb)
        cpa.wait()
        cpb.wait()
        for g in range(NGROUP):
            rows = rows0 + (g * 16)

            def ebody(hh, accs):
                out = []
                for k in range(4):
                    h = hh * 4 + k
                    col = jnp.full((16,), h, I32)
                    a = plsc.load_gather(arows, [rows, col])
                    b = plsc.load_gather(brows, [rows, col])
                    m = a + b
                    m = jnp.where(m > 0, m, 0.2 * m)
                    out.append(accs[k] + attv[h] * m)
                return tuple(out)

            accs = lax.fori_loop(0, DH // 4, ebody, (z16, z16, z16, z16))
            ex = jnp.exp((accs[0] + accs[1]) + (accs[2] + accs[3]))
            plsc.store_scatter(wbuf, [rows, colden], ex)

            def wbody(hh, _):
                for k in range(4):
                    h = hh * 4 + k
                    col = jnp.full((16,), h, I32)
                    a = plsc.load_gather(arows, [rows, col])
                    plsc.store_scatter(wbuf, [rows, col], a * ex)
                return 0

            lax.fori_loop(0, DH // 4, wbody, 0)
        pltpu.sync_copy(wbuf, accsh.at[dstv.at[j]], add=True)
        return 0

    lax.fori_loop(0, NCHUNK, chunk, 0)
    plsc.subcore_barrier()
    pltpu.sync_copy(accsh.at[pl.ds(sid * RPT, RPT)],
                    out_hbm.at[cid, pl.ds(sid * RPT, RPT)])


_edge = functools.partial(
    pl.kernel,
    out_type=jax.ShapeDtypeStruct((NC, N_NODE, ACC_W), F32),
    mesh=_MESH,
    scratch_types=[
        pltpu.VMEM((NCHUNK, CHUNK), I32),
        pltpu.VMEM((NCHUNK, CHUNK), I32),
        pltpu.VMEM((DH,), F32),
        pltpu.VMEM((CHUNK, DH), F32),
        pltpu.VMEM((CHUNK, DH), F32),
        pltpu.VMEM((CHUNK, ACC_W), F32),
        pltpu.VMEM((NCHUNK, ACC_W), F32),
        pltpu.VMEM_SHARED((N_NODE, ACC_W), F32),
        pltpu.SemaphoreType.DMA,
        pltpu.SemaphoreType.DMA,
    ],
)(_edge_body)


def _dec_body(z_hbm, ia_hbm, ib_hbm, out_hbm,
              iav, ibv, arows, brows, outv, sema, semb):
    cid = lax.axis_index("c")
    sid = lax.axis_index("s")
    wid = cid * NS + sid

    pltpu.sync_copy(ia_hbm.at[wid], iav)
    pltpu.sync_copy(ib_hbm.at[wid], ibv)

    rows0 = lax.iota(I32, 16)
    z16 = jnp.zeros((16,), F32)

    def chunk(j, _):
        cpa = pltpu.async_copy(z_hbm.at[iav.at[j]], arows, sema)
        cpb = pltpu.async_copy(z_hbm.at[ibv.at[j]], brows, semb)
        cpa.wait()
        cpb.wait()
        for g in range(NGROUP):
            rows = rows0 + (g * 16)

            def ebody(hh, accs):
                out = []
                for k in range(4):
                    h = hh * 4 + k
                    col = jnp.full((16,), h, I32)
                    a = plsc.load_gather(arows, [rows, col])
                    b = plsc.load_gather(brows, [rows, col])
                    out.append(accs[k] + a * b)
                return tuple(out)

            accs = lax.fori_loop(0, DH // 4, ebody, (z16, z16, z16, z16))
            e = (accs[0] + accs[1]) + (accs[2] + accs[3])
            outv[pl.ds(j * CHUNK + g * 16, 16)] = e
        return 0

    lax.fori_loop(0, DEC_NCH, chunk, 0)
    pltpu.sync_copy(outv, out_hbm.at[pl.ds(wid * DEC_PT, DEC_PT)])


_decoder = functools.partial(
    pl.kernel,
    out_type=jax.ShapeDtypeStruct((L_PAD,), F32),
    mesh=_MESH,
    scratch_types=[
        pltpu.VMEM((DEC_NCH, CHUNK), I32),
        pltpu.VMEM((DEC_NCH, CHUNK), I32),
        pltpu.VMEM((CHUNK, DH), F32),
        pltpu.VMEM((CHUNK, DH), F32),
        pltpu.VMEM((DEC_PT,), F32),
        pltpu.SemaphoreType.DMA,
        pltpu.SemaphoreType.DMA,
    ],
)(_dec_body)


def kernel(x_author, x_article, params, edge_index_publishes,
           edge_index_co_authors, edge_index_rev_publishes,
           edge_label_index):
    p = params

    def esplit(ei):
        return (ei[0].reshape(NW, NCHUNK, CHUNK),
                ei[1].reshape(NW, NCHUNK, CHUNK))

    sp, dp = esplit(edge_index_publishes)
    sc_, dc_ = esplit(edge_index_co_authors)
    sr, dr = esplit(edge_index_rev_publishes)

    xlp, xrp, xlc, xrc, xlr, xrr = _proj6(
        x_author, x_article,
        [p['Wl_pub0'], p['Wr_pub0'], p['Wl_co0'], p['Wr_co0'],
         p['Wl_rev0'], p['Wr_rev0']])
    part_pub = _edge(xlp, xrp, sp, dp, p['att_pub0'])
    part_co = _edge(xlc, xrc, sc_, dc_, p['att_co0'])
    part_rev = _edge(xlr, xrr, sr, dr, p['att_rev0'])

    h_art, psa, pssa = _c1_one(part_pub, p['b_pub0'],
                               p['linW_art0'], p['linb_art0'])
    h_auth, psu, pssu = _c1_two(part_co, part_rev, p['b_co0'], p['b_rev0'],
                                p['linW_auth0'], p['linb_auth0'])

    (xlr1,) = _c3(h_art, psa, pssa, p['bng_art0'], p['bnb_art0'],
                  [p['Wl_rev1']])
    xlc1, xrc1, xrr1 = _c3(h_auth, psu, pssu, p['bng_auth0'], p['bnb_auth0'],
                           [p['Wl_co1'], p['Wr_co1'], p['Wr_rev1']])

    part_co1 = _edge(xlc1, xrc1, sc_, dc_, p['att_co1'])
    part_rev1 = _edge(xlr1, xrr1, sr, dr, p['att_rev1'])

    h1, ps1, pss1 = _c1_two(part_co1, part_rev1, p['b_co1'], p['b_rev1'],
                            p['linW_auth1'], p['linb_auth1'])
    (z,) = _c3(h1, ps1, pss1, p['bng_auth1'], p['bnb_auth1'], [])

    n_lab = edge_label_index.shape[1]
    pad = L_PAD - n_lab
    eli = jnp.concatenate(
        [edge_label_index,
         jnp.zeros((2, pad), edge_label_index.dtype)], axis=1)
    ia = eli[0].reshape(NW, DEC_NCH, CHUNK)
    ib = eli[1].reshape(NW, DEC_NCH, CHUNK)
    out = _decoder(z, ia, ib)
    return out[:n_lab]


# speedup vs baseline: 8.9401x; 8.9401x over previous
import functools

import jax
import jax.numpy as jnp
from jax import lax
from jax.experimental import pallas as pl
from jax.experimental.pallas import tpu as pltpu
from jax.experimental.pallas import tpu_sc as plsc

F32 = jnp.float32
I32 = jnp.int32

N_NODE = 10000
DH = 128
NE = 320000
NC, NS = 2, 16
NW = NC * NS
EPT = NE // NW
CHUNK = 80
NCHUNK = EPT // CHUNK
NB = 5
NBLK = NCHUNK // NB
NGROUP = CHUNK // 16
N_PAD = 10240
DEN_ROWS = N_PAD // DH
N_ACC = N_PAD + DEN_ROWS
RPT = N_PAD // NS

L_PAD = 102400
DEC_PT = L_PAD // NW
DEC_NCH = DEC_PT // CHUNK
DEC_NBLK = DEC_NCH // NB

BR = 1000
GRID = N_NODE // BR

_EPS = 1e-16


def _dot(x, w):
    return jnp.dot(x, w, preferred_element_type=F32)


def _proj6_body(xa, xart, wlp, wrp, wlc, wrc, wlr, wrr,
                olp, orp, olc, orc, olr, orr):
    a = xa[...]
    r = xart[...]
    olp[...] = _dot(a, wlp[...])
    orp[...] = _dot(r, wrp[...])
    olc[...] = _dot(a, wlc[...])
    orc[...] = _dot(a, wrc[...])
    olr[...] = _dot(r, wlr[...])
    orr[...] = _dot(a, wrr[...])


def _proj6(xa, xart, ws):
    row = pl.BlockSpec((BR, DH), lambda i: (i, 0))
    wspec = pl.BlockSpec((DH, DH), lambda i: (0, 0))
    out = jax.ShapeDtypeStruct((N_NODE, DH), F32)
    return pl.pallas_call(
        _proj6_body,
        grid=(GRID,),
        in_specs=[row, row] + [wspec] * 6,
        out_specs=[row] * 6,
        out_shape=[out] * 6,
    )(xa, xart, *ws)


def _gat_out(num, den4, b):
    n = num[0] + num[1]
    d = den4[0, 0, 0] + den4[1, 0, 0]
    return n / (d + _EPS)[:, None] + b


def _c1_one_body(pp, dd, b, lw, lb, h_ref, ps, pss):
    g = _gat_out(pp[...], dd[...], b[...])
    h = g + _dot(g, lw[...]) + lb[...]
    h_ref[...] = h
    ps[...] = jnp.sum(h, axis=0).reshape(1, 1, DH)
    pss[...] = jnp.sum(h * h, axis=0).reshape(1, 1, DH)


def _c1_two_body(p1, p2, d1, d2, b1, b2, lw, lb, h_ref, ps, pss):
    g = _gat_out(p1[...], d1[...], b1[...]) + _gat_out(p2[...], d2[...],
                                                      b2[...])
    h = g + _dot(g, lw[...]) + lb[...]
    h_ref[...] = h
    ps[...] = jnp.sum(h, axis=0).reshape(1, 1, DH)
    pss[...] = jnp.sum(h * h, axis=0).reshape(1, 1, DH)


def _c1_specs(nparts):
    pspec = pl.BlockSpec((NC, BR, DH), lambda i: (0, i, 0))
    dspec = pl.BlockSpec((NC, 1, 1, BR), lambda i: (0, i, 0, 0))
    vspec = pl.BlockSpec((DH,), lambda i: (0,))
    wspec = pl.BlockSpec((DH, DH), lambda i: (0, 0))
    row = pl.BlockSpec((BR, DH), lambda i: (i, 0))
    st = pl.BlockSpec((1, 1, DH), lambda i: (i, 0, 0))
    in_specs = ([pspec] * nparts + [dspec] * nparts + [vspec] * nparts
                + [wspec, vspec])
    out_specs = [row, st, st]
    out_shape = [jax.ShapeDtypeStruct((N_NODE, DH), F32),
                 jax.ShapeDtypeStruct((GRID, 1, DH), F32),
                 jax.ShapeDtypeStruct((GRID, 1, DH), F32)]
    return in_specs, out_specs, out_shape


def _c1_one(part, den4, b, lw, lb):
    i, o, s = _c1_specs(1)
    return pl.pallas_call(_c1_one_body, grid=(GRID,), in_specs=i,
                          out_specs=o, out_shape=s)(part, den4, b, lw, lb)


def _c1_two(p1, p2, d1, d2, b1, b2, lw, lb):
    i, o, s = _c1_specs(2)
    return pl.pallas_call(_c1_two_body, grid=(GRID,), in_specs=i,
                          out_specs=o, out_shape=s)(p1, p2, d1, d2, b1, b2,
                                                    lw, lb)


def _bn_apply(h, ps, pss, g, b):
    mu = jnp.sum(ps[...], axis=(0, 1)) * (1.0 / N_NODE)
    msq = jnp.sum(pss[...], axis=(0, 1)) * (1.0 / N_NODE)
    var = msq - mu * mu
    scale = g[...] * lax.rsqrt(var + 1e-5)
    shift = b[...] - mu * scale
    return h[...] * scale + shift


def _c3_body(h, ps, pss, g, b, *rest):
    nw = len(rest) // 2
    y = _bn_apply(h, ps, pss, g, b)
    if nw:
        y = jnp.maximum(y, 0.0)
        for k in range(nw):
            rest[nw + k][...] = _dot(y, rest[k][...])
    else:
        rest[0][...] = y


def _c3(h, ps, pss, g, b, ws):
    nw = len(ws)
    row = pl.BlockSpec((BR, DH), lambda i: (i, 0))
    st = pl.BlockSpec((GRID, 1, DH), lambda i: (0, 0, 0))
    vspec = pl.BlockSpec((DH,), lambda i: (0,))
    wspec = pl.BlockSpec((DH, DH), lambda i: (0, 0))
    nout = max(nw, 1)
    return pl.pallas_call(
        _c3_body,
        grid=(GRID,),
        in_specs=[row, st, st, vspec, vspec] + [wspec] * nw,
        out_specs=[row] * nout,
        out_shape=[jax.ShapeDtypeStruct((N_NODE, DH), F32)] * nout,
    )(h, ps, pss, g, b, *ws)


_MESH = plsc.VectorSubcoreMesh(core_axis_name="c", subcore_axis_name="s")


def _edge_body(xl_hbm, xr_hbm, src_hbm, dst_hbm, att_hbm, out_hbm,
               srcb, dstb, attv, arows, brows, denv, rowidx, zerov,
               accsh, sema, semb):
    cid = lax.axis_index("c")
    sid = lax.axis_index("s")
    wid = cid * NS + sid

    pltpu.sync_copy(att_hbm, attv)

    z16 = jnp.zeros((16,), F32)
    rows0 = lax.iota(I32, 16)

    def zb(r, _):
        for c in range(DH // 16):
            zerov[r, pl.ds(c * 16, 16)] = z16
        return 0
    lax.fori_loop(0, 8, zb, 0)

    def zd(r, _):
        for c in range(DH // 16):
            denv[r, pl.ds(c * 16, 16)] = z16
        return 0
    lax.fori_loop(0, DEN_ROWS, zd, 0)

    for i in range(DEN_ROWS // 16):
        rowidx[pl.ds(i * 16, 16)] = rows0 + (N_PAD + i * 16)

    for k in range(RPT // 8):
        pltpu.sync_copy(zerov, accsh.at[pl.ds(sid * RPT + k * 8, 8)])

    @pl.when(sid == 0)
    def _():
        for k in range(DEN_ROWS // 8):
            pltpu.sync_copy(zerov, accsh.at[pl.ds(N_PAD + k * 8, 8)])
    plsc.subcore_barrier()

    attc = [attv[pl.ds(c * 16, 16)] for c in range(DH // 16)]

    def block(blk, _):
        pltpu.sync_copy(src_hbm.at[wid, blk], srcb)
        pltpu.sync_copy(dst_hbm.at[wid, blk], dstb)

        def chunk(jj, _):
            cpa = pltpu.async_copy(xl_hbm.at[srcb.at[jj]], arows, sema)
            cpb = pltpu.async_copy(xr_hbm.at[dstb.at[jj]], brows, semb)
            cpa.wait()
            cpb.wait()
            for g in range(NGROUP):
                dst16 = dstb[jj, pl.ds(g * 16, 16)]

                def kbody(k, exvec):
                    r = g * 16 + k
                    acc = z16
                    avs = []
                    for c in range(DH // 16):
                        a = arows[r, pl.ds(c * 16, 16)]
                        b = brows[r, pl.ds(c * 16, 16)]
                        m = a + b
                        m = jnp.where(m > 0, m, 0.2 * m)
                        acc = acc + attc[c] * m
                        avs.append(a)
                    e = jnp.sum(acc)
                    ex = jnp.exp(jnp.full((16,), e, F32))
                    for c in range(DH // 16):
                        arows[r, pl.ds(c * 16, 16)] = avs[c] * ex
                    return jnp.where(rows0 == k, ex, exvec)

                exvec = lax.fori_loop(0, 16, kbody, z16)
                plsc.addupdate_scatter(
                    denv,
                    [lax.shift_right_logical(dst16, 7),
                     lax.bitwise_and(dst16, 127)],
                    exvec)
            pltpu.sync_copy(arows, accsh.at[dstb.at[jj]], add=True)
            return 0

        lax.fori_loop(0, NB, chunk, 0)
        return 0

    lax.fori_loop(0, NBLK, block, 0)
    pltpu.sync_copy(denv, accsh.at[rowidx], add=True)
    plsc.subcore_barrier()
    pltpu.sync_copy(accsh.at[pl.ds(sid * RPT, RPT)],
                    out_hbm.at[cid, pl.ds(sid * RPT, RPT)])

    @pl.when(sid < DEN_ROWS // 8)
    def _():
        pltpu.sync_copy(
            accsh.at[pl.ds(N_PAD + sid * 8, 8)],
            out_hbm.at[cid, pl.ds(N_PAD + sid * 8, 8)])


_edge = functools.partial(
    pl.kernel,
    out_type=jax.ShapeDtypeStruct((NC, N_ACC, DH), F32),
    mesh=_MESH,
    compiler_params=pltpu.CompilerParams(needs_layout_passes=False),
    scratch_types=[
        pltpu.VMEM((NB, CHUNK), I32),
        pltpu.VMEM((NB, CHUNK), I32),
        pltpu.VMEM((DH,), F32),
        pltpu.VMEM((CHUNK, DH), F32),
        pltpu.VMEM((CHUNK, DH), F32),
        pltpu.VMEM((DEN_ROWS, DH), F32),
        pltpu.VMEM((DEN_ROWS,), I32),
        pltpu.VMEM((8, DH), F32),
        pltpu.VMEM_SHARED((N_ACC, DH), F32),
        pltpu.SemaphoreType.DMA,
        pltpu.SemaphoreType.DMA,
    ],
)(_edge_body)


def _dec_body(z_hbm, ia_hbm, ib_hbm, out_hbm,
              iab, ibb, arows, brows, outv, sema, semb):
    cid = lax.axis_index("c")
    sid = lax.axis_index("s")
    wid = cid * NS + sid

    rows0 = lax.iota(I32, 16)
    z16 = jnp.zeros((16,), F32)

    def block(blk, _):
        pltpu.sync_copy(ia_hbm.at[wid, blk], iab)
        pltpu.sync_copy(ib_hbm.at[wid, blk], ibb)

        def chunk(jj, _):
            cpa = pltpu.async_copy(z_hbm.at[iab.at[jj]], arows, sema)
            cpb = pltpu.async_copy(z_hbm.at[ibb.at[jj]], brows, semb)
            cpa.wait()
            cpb.wait()
            for g in range(NGROUP):

                def kbody(k, evec):
                    r = g * 16 + k
                    acc = z16
                    for c in range(DH // 16):
                        a = arows[r, pl.ds(c * 16, 16)]
                        b = brows[r, pl.ds(c * 16, 16)]
                        acc = acc + a * b
                    e = jnp.sum(acc)
                    return jnp.where(rows0 == k, jnp.full((16,), e, F32),
                                     evec)

                evec = lax.fori_loop(0, 16, kbody, z16)
                outv[pl.ds((blk * NB + jj) * CHUNK + g * 16, 16)] = evec
            return 0

        lax.fori_loop(0, NB, chunk, 0)
        return 0

    lax.fori_loop(0, DEC_NBLK, block, 0)
    pltpu.sync_copy(outv, out_hbm.at[pl.ds(wid * DEC_PT, DEC_PT)])


_decoder = functools.partial(
    pl.kernel,
    out_type=jax.ShapeDtypeStruct((L_PAD,), F32),
    mesh=_MESH,
    compiler_params=pltpu.CompilerParams(needs_layout_passes=False),
    scratch_types=[
        pltpu.VMEM((NB, CHUNK), I32),
        pltpu.VMEM((NB, CHUNK), I32),
        pltpu.VMEM((CHUNK, DH), F32),
        pltpu.VMEM((CHUNK, DH), F32),
        pltpu.VMEM((DEC_PT,), F32),
        pltpu.SemaphoreType.DMA,
        pltpu.SemaphoreType.DMA,
    ],
)(_dec_body)


def _split_acc(acc):
    num = acc[:, :N_PAD, :]
    den = acc[:, N_PAD:, :].reshape(NC, N_PAD)[:, :N_NODE]
    den4 = den.reshape(NC, GRID, 1, BR)
    return num, den4


def kernel(x_author, x_article, params, edge_index_publishes,
           edge_index_co_authors, edge_index_rev_publishes,
           edge_label_index):
    p = params

    def esplit(ei):
        return (ei[0].reshape(NW, NBLK, NB, CHUNK),
                ei[1].reshape(NW, NBLK, NB, CHUNK))

    sp, dp = esplit(edge_index_publishes)
    sc_, dc_ = esplit(edge_index_co_authors)
    sr, dr = esplit(edge_index_rev_publishes)

    xlp, xrp, xlc, xrc, xlr, xrr = _proj6(
        x_author, x_article,
        [p['Wl_pub0'], p['Wr_pub0'], p['Wl_co0'], p['Wr_co0'],
         p['Wl_rev0'], p['Wr_rev0']])
    np_, d4p = _split_acc(_edge(xlp, xrp, sp, dp, p['att_pub0']))
    nc_, d4c = _split_acc(_edge(xlc, xrc, sc_, dc_, p['att_co0']))
    nr_, d4r = _split_acc(_edge(xlr, xrr, sr, dr, p['att_rev0']))

    h_art, psa, pssa = _c1_one(np_, d4p, p['b_pub0'],
                               p['linW_art0'], p['linb_art0'])
    h_auth, psu, pssu = _c1_two(nc_, nr_, d4c, d4r, p['b_co0'], p['b_rev0'],
                                p['linW_auth0'], p['linb_auth0'])

    (xlr1,) = _c3(h_art, psa, pssa, p['bng_art0'], p['bnb_art0'],
                  [p['Wl_rev1']])
    xlc1, xrc1, xrr1 = _c3(h_auth, psu, pssu, p['bng_auth0'], p['bnb_auth0'],
                           [p['Wl_co1'], p['Wr_co1'], p['Wr_rev1']])

    nc1, d4c1 = _split_acc(_edge(xlc1, xrc1, sc_, dc_, p['att_co1']))
    nr1, d4r1 = _split_acc(_edge(xlr1, xrr1, sr, dr, p['att_rev1']))

    h1, ps1, pss1 = _c1_two(nc1, nr1, d4c1, d4r1, p['b_co1'], p['b_rev1'],
                            p['linW_auth1'], p['linb_auth1'])
    (z,) = _c3(h1, ps1, pss1, p['bng_auth1'], p['bnb_auth1'], [])

    n_lab = edge_label_index.shape[1]
    pad = L_PAD - n_lab
    eli = jnp.concatenate(
        [edge_label_index,
         jnp.zeros((2, pad), edge_label_index.dtype)], axis=1)
    ia = eli[0].reshape(NW, DEC_NBLK, NB, CHUNK)
    ib = eli[1].reshape(NW, DEC_NBLK, NB, CHUNK)
    out = _decoder(z, ia, ib)
    return out[:n_lab]
